# Initial kernel scaffold; baseline (speedup 1.0000x reference)
import jax
import jax.numpy as jnp
from jax.experimental import pallas as pl


def kernel(h, pos, edge_index, edge_attr, ew1, eb1, ew2, eb2, nw1, nb1, nw2, nb2, cw1, cb1, cw2, cb2, ln_g, ln_b):
    raise NotImplementedError("write your pallas kernel here")


# speedup vs baseline: 3.0034x; 3.0034x over previous
import functools

import jax
import jax.numpy as jnp
from jax import lax
from jax.experimental import pallas as pl
from jax.experimental.pallas import tpu as pltpu
from jax.experimental.pallas import tpu_sc as plsc

N = 10000
E = 320000
H = 128
ED = 16

NC = 2
NS = 16
NW = NC * NS
CHUNK = 256
NCHUNK = E // CHUNK
KMAX = -(-NCHUNK // NW)
IDXROWS = CHUNK // 128
NPT = N // NS
SCHUNK = 128
SNCHUNK = E // SCHUNK
SKMAX = -(-SNCHUNK // NW)


def _gather_body(hA, hB, pos8, row2d, col2d, outA, outB, outR, outC,
                 idr, idc, bufA, bufB, bufR, bufC, sem, semw):
    c = lax.axis_index("c")
    s = lax.axis_index("s")
    wid = s * NC + c

    def body(k, _):
        g = k * NW + wid

        @pl.when(g < NCHUNK)
        def _():
            e0 = g * CHUNK
            r0 = g * IDXROWS
            pltpu.sync_copy(row2d.at[pl.ds(r0, IDXROWS)], idr)
            pltpu.sync_copy(col2d.at[pl.ds(r0, IDXROWS)], idc)
            descs = []
            for j in range(IDXROWS):
                sl = pl.ds(j * 128, 128)
                descs.append(pltpu.async_copy(hA.at[idr.at[j]], bufA.at[sl], sem))
                descs.append(pltpu.async_copy(hB.at[idc.at[j]], bufB.at[sl], sem))
                descs.append(pltpu.async_copy(pos8.at[idr.at[j]], bufR.at[sl], sem))
                descs.append(pltpu.async_copy(pos8.at[idc.at[j]], bufC.at[sl], sem))
            for d in descs:
                d.wait()
            wds = [
                pltpu.async_copy(bufA, outA.at[pl.ds(e0, CHUNK)], semw),
                pltpu.async_copy(bufB, outB.at[pl.ds(e0, CHUNK)], semw),
                pltpu.async_copy(bufR, outR.at[pl.ds(e0, CHUNK)], semw),
                pltpu.async_copy(bufC, outC.at[pl.ds(e0, CHUNK)], semw),
            ]
            for d in wds:
                d.wait()
        return 0

    lax.fori_loop(0, KMAX, body, 0)


def _scatter_body(m, t8, row2d, zM, z2, aggM, agg2,
                  idr, mbuf, tbuf, sharedM, shared2):
    c = lax.axis_index("c")
    s = lax.axis_index("s")
    wid = s * NC + c
    row0 = s * NPT

    pltpu.sync_copy(zM, sharedM.at[pl.ds(row0, NPT)])
    pltpu.sync_copy(z2, shared2.at[pl.ds(row0, NPT)])
    plsc.subcore_barrier()

    def body(k, _):
        g = k * NW + wid

        @pl.when(g < SNCHUNK)
        def _():
            e0 = g * SCHUNK
            pltpu.sync_copy(row2d.at[pl.ds(g, 1)], idr)
            pltpu.sync_copy(m.at[pl.ds(e0, SCHUNK)], mbuf)
            pltpu.sync_copy(t8.at[pl.ds(e0, SCHUNK)], tbuf)
            pltpu.sync_copy(mbuf, sharedM.at[idr.at[0]], add=True)
            pltpu.sync_copy(tbuf, shared2.at[idr.at[0]], add=True)
        return 0

    lax.fori_loop(0, SKMAX, body, 0)
    plsc.subcore_barrier()
    pltpu.sync_copy(sharedM.at[pl.ds(row0, NPT)], aggM.at[c, pl.ds(row0, NPT)])
    pltpu.sync_copy(shared2.at[pl.ds(row0, NPT)], agg2.at[c, pl.ds(row0, NPT)])


def _prep_tc(h_ref, wa_ref, wb_ref, ha_ref, hb_ref):
    h = h_ref[...]
    ha_ref[...] = jnp.dot(h, wa_ref[...], preferred_element_type=jnp.float32)
    hb_ref[...] = jnp.dot(h, wb_ref[...], preferred_element_type=jnp.float32)


def _edge_tc(a_ref, b_ref, pr_ref, pc_ref, ea_ref,
             w1c_ref, w1d_ref, eb1_ref, ew2_ref, eb2_ref,
             cw1_ref, cb1_ref, cw2_ref, cb2_ref,
             m_ref, t8_ref):
    rad = pr_ref[...] - pc_ref[...]
    pre = (a_ref[...] + b_ref[...]
           + jnp.dot(rad, w1c_ref[...], preferred_element_type=jnp.float32)
           + jnp.dot(ea_ref[...], w1d_ref[...], preferred_element_type=jnp.float32)
           + eb1_ref[...])
    u = pre * jax.nn.sigmoid(pre)
    msg = jnp.dot(u, ew2_ref[...], preferred_element_type=jnp.float32) + eb2_ref[...]
    m_ref[...] = msg
    cm = jnp.dot(msg, cw1_ref[...], preferred_element_type=jnp.float32) + cb1_ref[...]
    cm = cm * jax.nn.sigmoid(cm)
    cw = jnp.dot(cm, cw2_ref[...], preferred_element_type=jnp.float32) + cb2_ref[...]
    coord_w = cw[:, 0:1]
    lane = lax.broadcasted_iota(jnp.int32, rad.shape, 1)
    t8_ref[...] = rad * coord_w + jnp.where(lane == 3, 1.0, 0.0)


def _node_tc(h_ref, pos_ref, aggA_ref, aggB_ref, t2a_ref, t2b_ref,
             n1a_ref, n1b_ref, nb1_ref, nw2_ref, nb2_ref, g_ref, b_ref,
             hout_ref, pout_ref):
    h = h_ref[...]
    agg = aggA_ref[...] + aggB_ref[...]
    z = (jnp.dot(h, n1a_ref[...], preferred_element_type=jnp.float32)
         + jnp.dot(agg, n1b_ref[...], preferred_element_type=jnp.float32)
         + nb1_ref[...])
    u = z * jax.nn.sigmoid(z)
    h_upd = jnp.dot(u, nw2_ref[...], preferred_element_type=jnp.float32) + nb2_ref[...]
    hn = h + h_upd
    mean = jnp.mean(hn, axis=1, keepdims=True)
    d = hn - mean
    var = jnp.mean(d * d, axis=1, keepdims=True)
    hout_ref[...] = d * lax.rsqrt(var + 1e-5) * g_ref[...] + b_ref[...]

    t = t2a_ref[...] + t2b_ref[...]
    deg = jnp.maximum(t[:, 3:4], 1.0)
    pout_ref[...] = pos_ref[...] + t / deg


def kernel(h, pos, edge_index, edge_attr, ew1, eb1, ew2, eb2,
           nw1, nb1, nw2, nb2, cw1, cb1, cw2, cb2, ln_g, ln_b):
    f32 = jnp.float32
    row2d = edge_index[0].reshape(E // 128, 128)
    col2d = edge_index[1].reshape(E // 128, 128)
    pos8 = jnp.pad(pos, ((0, 0), (0, 5)))

    wa = ew1[:H]
    wb = ew1[H:2 * H]
    BN = 1000
    ha, hb = pl.pallas_call(
        _prep_tc,
        grid=(N // BN,),
        in_specs=[
            pl.BlockSpec((BN, H), lambda i: (i, 0)),
            pl.BlockSpec((H, H), lambda i: (0, 0)),
            pl.BlockSpec((H, H), lambda i: (0, 0)),
        ],
        out_specs=[
            pl.BlockSpec((BN, H), lambda i: (i, 0)),
            pl.BlockSpec((BN, H), lambda i: (i, 0)),
        ],
        out_shape=[jax.ShapeDtypeStruct((N, H), f32)] * 2,
    )(h, wa, wb)

    mesh = plsc.VectorSubcoreMesh(core_axis_name="c", subcore_axis_name="s")
    gath = pl.kernel(
        _gather_body,
        out_type=[
            jax.ShapeDtypeStruct((E, H), f32),
            jax.ShapeDtypeStruct((E, H), f32),
            jax.ShapeDtypeStruct((E, 8), f32),
            jax.ShapeDtypeStruct((E, 8), f32),
        ],
        mesh=mesh,
        scratch_types=[
            pltpu.VMEM((IDXROWS, 128), jnp.int32),
            pltpu.VMEM((IDXROWS, 128), jnp.int32),
            pltpu.VMEM((CHUNK, H), f32),
            pltpu.VMEM((CHUNK, H), f32),
            pltpu.VMEM((CHUNK, 8), f32),
            pltpu.VMEM((CHUNK, 8), f32),
            pltpu.SemaphoreType.DMA,
            pltpu.SemaphoreType.DMA,
        ],
        compiler_params=pltpu.CompilerParams(use_tc_tiling_on_sc=False),
    )
    a_g, b_g, pr_g, pc_g = gath(ha, hb, pos8, row2d, col2d)

    w1c = jnp.pad(ew1[2 * H:2 * H + 3], ((0, 5), (0, 0)))
    w1d = ew1[2 * H + 3:]
    BE = 512
    m_e, t8_e = pl.pallas_call(
        _edge_tc,
        grid=(E // BE,),
        in_specs=[
            pl.BlockSpec((BE, H), lambda i: (i, 0)),
            pl.BlockSpec((BE, H), lambda i: (i, 0)),
            pl.BlockSpec((BE, 8), lambda i: (i, 0)),
            pl.BlockSpec((BE, 8), lambda i: (i, 0)),
            pl.BlockSpec((BE, ED), lambda i: (i, 0)),
            pl.BlockSpec((8, H), lambda i: (0, 0)),
            pl.BlockSpec((ED, H), lambda i: (0, 0)),
            pl.BlockSpec((H,), lambda i: (0,)),
            pl.BlockSpec((H, H), lambda i: (0, 0)),
            pl.BlockSpec((H,), lambda i: (0,)),
            pl.BlockSpec((H, H), lambda i: (0, 0)),
            pl.BlockSpec((H,), lambda i: (0,)),
            pl.BlockSpec((H, 1), lambda i: (0, 0)),
            pl.BlockSpec((1,), lambda i: (0,)),
        ],
        out_specs=[
            pl.BlockSpec((BE, H), lambda i: (i, 0)),
            pl.BlockSpec((BE, 8), lambda i: (i, 0)),
        ],
        out_shape=[
            jax.ShapeDtypeStruct((E, H), f32),
            jax.ShapeDtypeStruct((E, 8), f32),
        ],
    )(a_g, b_g, pr_g, pc_g, edge_attr, w1c, w1d, eb1, ew2, eb2,
      cw1, cb1, cw2, cb2)

    zM = jnp.zeros((NPT, H), f32)
    z2 = jnp.zeros((NPT, 8), f32)
    scat = pl.kernel(
        _scatter_body,
        out_type=[
            jax.ShapeDtypeStruct((NC, N, H), f32),
            jax.ShapeDtypeStruct((NC, N, 8), f32),
        ],
        mesh=mesh,
        scratch_types=[
            pltpu.VMEM((1, 128), jnp.int32),
            pltpu.VMEM((SCHUNK, H), f32),
            pltpu.VMEM((SCHUNK, 8), f32),
            pltpu.VMEM_SHARED((N, H), f32),
            pltpu.VMEM_SHARED((N, 8), f32),
        ],
        compiler_params=pltpu.CompilerParams(use_tc_tiling_on_sc=False),
    )
    aggM, agg2 = scat(m_e, t8_e, row2d, zM, z2)

    n1a = nw1[:H]
    n1b = nw1[H:]
    BNo = 1000
    h_out, pos8_out = pl.pallas_call(
        _node_tc,
        grid=(N // BNo,),
        in_specs=[
            pl.BlockSpec((BNo, H), lambda i: (i, 0)),
            pl.BlockSpec((BNo, 8), lambda i: (i, 0)),
            pl.BlockSpec((BNo, H), lambda i: (i, 0)),
            pl.BlockSpec((BNo, H), lambda i: (i, 0)),
            pl.BlockSpec((BNo, 8), lambda i: (i, 0)),
            pl.BlockSpec((BNo, 8), lambda i: (i, 0)),
            pl.BlockSpec((H, H), lambda i: (0, 0)),
            pl.BlockSpec((H, H), lambda i: (0, 0)),
            pl.BlockSpec((H,), lambda i: (0,)),
            pl.BlockSpec((H, H), lambda i: (0, 0)),
            pl.BlockSpec((H,), lambda i: (0,)),
            pl.BlockSpec((H,), lambda i: (0,)),
            pl.BlockSpec((H,), lambda i: (0,)),
        ],
        out_specs=[
            pl.BlockSpec((BNo, H), lambda i: (i, 0)),
            pl.BlockSpec((BNo, 8), lambda i: (i, 0)),
        ],
        out_shape=[
            jax.ShapeDtypeStruct((N, H), f32),
            jax.ShapeDtypeStruct((N, 8), f32),
        ],
    )(h, pos8, aggM[0], aggM[1], agg2[0], agg2[1],
      n1a, n1b, nb1, nw2, nb2, ln_g, ln_b)

    return (h_out, pos8_out[:, :3])
